# baseline (speedup 1.0000x reference)
import functools

import jax
import jax.numpy as jnp
from jax.experimental import pallas as pl
from jax.experimental.pallas import tpu as pltpu
from jax.experimental.shard_map import shard_map
from jax.sharding import Mesh, NamedSharding, PartitionSpec as P


def _moe_tile(u_ref, wg_ref, w1s_ref, b1s_ref, w2s_ref, w1_ref, b1_ref,
              w2r_ref, b2all_ref, out_ref, h_ref, *, n_shared, n_routed):
    steps = n_shared + n_routed
    u = u_ref[...]
    tt = u.shape[0]
    hh = w1s_ref.shape[-1]

    logits = jax.lax.dot_general(
        u, wg_ref[...], (((1,), (0,)), ((), ())),
        preferred_element_type=jnp.float32)
    m = jnp.min(logits, axis=-1, keepdims=True)
    idx = jax.lax.broadcasted_iota(jnp.int32, logits.shape, 1)
    excl = jnp.max(jnp.where(logits <= m, idx, -1), axis=-1, keepdims=True)
    keep = idx != excl
    mx = jnp.max(logits, axis=-1, keepdims=True)
    ex = jnp.where(keep, jnp.exp(logits - mx), 0.0)
    sm = ex / jnp.sum(ex, axis=-1, keepdims=True)
    lane = jax.lax.broadcasted_iota(jnp.int32, (tt, 16), 1)
    shared_w = jnp.where(lane < n_shared, 1.0 / n_shared, 0.0)
    routed_w = jnp.where(
        jnp.logical_and(lane >= n_shared, lane < steps),
        jnp.pad(sm, ((0, 0), (n_shared, 16 - steps))), 0.0)
    comb = shared_w + routed_w

    for s in range(steps):
        if s < n_shared:
            w1 = w1s_ref[s]
            b1 = b1s_ref[s]
        else:
            w1 = w1_ref[s - n_shared]
            b1 = b1_ref[s - n_shared]
        h = jax.lax.dot_general(
            u, w1, (((1,), (0,)), ((), ())),
            preferred_element_type=jnp.float32)
        w = comb[:, s:s + 1]
        h_ref[:, s * hh:(s + 1) * hh] = jnp.maximum(h + b1, 0.0) * w

    hs = h_ref[:, :n_shared * hh]
    hr = h_ref[:, n_shared * hh:]
    o = jax.lax.dot_general(
        hs, w2s_ref[0], (((1,), (0,)), ((), ())),
        preferred_element_type=jnp.float32)
    o = o + jax.lax.dot_general(
        hr, w2r_ref[...], (((1,), (0,)), ((), ())),
        preferred_element_type=jnp.float32)
    bias = jax.lax.dot_general(
        comb, b2all_ref[...], (((1,), (0,)), ((), ())),
        preferred_element_type=jnp.float32)
    out_ref[...] = o + u + bias


def _moe_local(u, Wg, W1s, b1s, W2s, b2s, W1, b1, W2, b2):
    T, Dm = u.shape
    n_shared = W1s.shape[0]
    n_routed = W1.shape[0]
    steps = n_shared + n_routed
    Hh = W1.shape[-1]
    TT = 256
    w2r = W2.reshape(n_routed * Hh, Dm)
    b1s3 = b1s[:, None, :]
    b13 = b1[:, None, :]
    b2all = jnp.pad(jnp.concatenate([b2s, b2], axis=0),
                    ((0, 16 - steps), (0, 0)))

    grid = (T // TT,)
    out = pl.pallas_call(
        functools.partial(_moe_tile, n_shared=n_shared, n_routed=n_routed),
        grid=grid,
        in_specs=[
            pl.BlockSpec((TT, Dm), lambda i: (i, 0)),
            pl.BlockSpec(Wg.shape, lambda i: (0, 0)),
            pl.BlockSpec(W1s.shape, lambda i: (0, 0, 0)),
            pl.BlockSpec((n_shared, 1, Hh), lambda i: (0, 0, 0)),
            pl.BlockSpec(W2s.shape, lambda i: (0, 0, 0)),
            pl.BlockSpec(W1.shape, lambda i: (0, 0, 0)),
            pl.BlockSpec((n_routed, 1, Hh), lambda i: (0, 0, 0)),
            pl.BlockSpec((n_routed * Hh, Dm), lambda i: (0, 0)),
            pl.BlockSpec((16, Dm), lambda i: (0, 0)),
        ],
        out_specs=pl.BlockSpec((TT, Dm), lambda i: (i, 0)),
        out_shape=jax.ShapeDtypeStruct((T, Dm), jnp.float32),
        scratch_shapes=[
            pltpu.VMEM((TT, steps * Hh), jnp.float32),
        ],
    )(u, Wg, W1s, b1s3, W2s, W1, b13, w2r, b2all)
    return out


_DEVS = jax.devices()
_NDEV = len(_DEVS) if 2048 % (len(_DEVS) * 256) == 0 else 1
_MESH = Mesh(_DEVS[:_NDEV], ("x",))
_TOK = NamedSharding(_MESH, P("x", None))
_REPL = NamedSharding(_MESH, P())


def kernel(u, Wg, W1s, b1s, W2s, b2s, W1, b1, W2, b2):
    local = shard_map(
        _moe_local, mesh=_MESH,
        in_specs=(P("x", None), P(), P(), P(), P(), P(), P(), P(), P(), P()),
        out_specs=P("x", None), check_rep=False)
    return local(u, Wg, W1s, b1s, W2s, b2s, W1, b1, W2, b2)


kernel = jax.jit(
    kernel,
    in_shardings=(_TOK, _REPL, _REPL, _REPL, _REPL, _REPL, _REPL, _REPL,
                  _REPL, _REPL),
    out_shardings=_TOK)


# speedup vs baseline: 7.6741x; 7.6741x over previous
import functools

import jax
import jax.numpy as jnp
from jax.experimental import pallas as pl
from jax.experimental.pallas import tpu as pltpu


def _moe_tile(u_ref, wg_ref, w1s_ref, b1s_ref, w2s_ref, b2s_ref, w1_ref,
              b1_ref, w2r_ref, b2_ref, out_ref, h_ref, *, n_shared,
              n_routed):
    u = u_ref[...]
    tt = u.shape[0]
    hh = w1s_ref.shape[-1]

    logits = jax.lax.dot_general(
        u, wg_ref[...], (((1,), (0,)), ((), ())),
        preferred_element_type=jnp.float32)
    m = jnp.min(logits, axis=-1, keepdims=True)
    idx = jax.lax.broadcasted_iota(jnp.int32, logits.shape, 1)
    excl = jnp.max(jnp.where(logits <= m, idx, -1), axis=-1, keepdims=True)
    keep = idx != excl
    mx = jnp.max(logits, axis=-1, keepdims=True)
    ex = jnp.where(keep, jnp.exp(logits - mx), 0.0)
    sm = ex / jnp.sum(ex, axis=-1, keepdims=True)

    for e in range(n_routed):
        h = jax.lax.dot_general(
            u, w1_ref[e], (((1,), (0,)), ((), ())),
            preferred_element_type=jnp.float32)
        h = jnp.maximum(h + b1_ref[e:e + 1, :], 0.0) * sm[:, e:e + 1]
        h_ref[:, e * hh:(e + 1) * hh] = h
    for s in range(n_shared):
        h = jax.lax.dot_general(
            u, w1s_ref[s], (((1,), (0,)), ((), ())),
            preferred_element_type=jnp.float32)
        h = jnp.maximum(h + b1s_ref[s:s + 1, :], 0.0) * (1.0 / n_shared)
        h_ref[:, (n_routed + s) * hh:(n_routed + s + 1) * hh] = h

    hr = h_ref[:, :n_routed * hh]
    o = jax.lax.dot_general(
        hr, w2r_ref[...], (((1,), (0,)), ((), ())),
        preferred_element_type=jnp.float32)
    for s in range(n_shared):
        o = o + jax.lax.dot_general(
            h_ref[:, (n_routed + s) * hh:(n_routed + s + 1) * hh],
            w2s_ref[s], (((1,), (0,)), ((), ())),
            preferred_element_type=jnp.float32)
    bias = jax.lax.dot_general(
        sm, b2_ref[...], (((1,), (0,)), ((), ())),
        preferred_element_type=jnp.float32)
    bias = bias + jnp.sum(b2s_ref[...], axis=0, keepdims=True) / n_shared
    out_ref[...] = o + u + bias


@functools.partial(jax.jit, static_argnames=())
def kernel(u, Wg, W1s, b1s, W2s, b2s, W1, b1, W2, b2):
    T, Dm = u.shape
    n_shared = W1s.shape[0]
    n_routed = W1.shape[0]
    steps = n_shared + n_routed
    Hh = W1.shape[-1]
    TT = 256
    w2r = W2.reshape(n_routed * Hh, Dm)

    grid = (T // TT,)
    out = pl.pallas_call(
        functools.partial(_moe_tile, n_shared=n_shared, n_routed=n_routed),
        grid=grid,
        in_specs=[
            pl.BlockSpec((TT, Dm), lambda i: (i, 0)),
            pl.BlockSpec(Wg.shape, lambda i: (0, 0)),
            pl.BlockSpec(W1s.shape, lambda i: (0, 0, 0)),
            pl.BlockSpec(b1s.shape, lambda i: (0, 0)),
            pl.BlockSpec(W2s.shape, lambda i: (0, 0, 0)),
            pl.BlockSpec(b2s.shape, lambda i: (0, 0)),
            pl.BlockSpec(W1.shape, lambda i: (0, 0, 0)),
            pl.BlockSpec(b1.shape, lambda i: (0, 0)),
            pl.BlockSpec((n_routed * Hh, Dm), lambda i: (0, 0)),
            pl.BlockSpec(b2.shape, lambda i: (0, 0)),
        ],
        out_specs=pl.BlockSpec((TT, Dm), lambda i: (i, 0)),
        out_shape=jax.ShapeDtypeStruct((T, Dm), jnp.float32),
        scratch_shapes=[
            pltpu.VMEM((TT, steps * Hh), jnp.float32),
        ],
    )(u, Wg, W1s, b1s, W2s, b2s, W1, b1, w2r, b2)
    return out
